# baseline (speedup 1.0000x reference)
import jax
import jax.numpy as jnp
from jax import lax
from jax.experimental import pallas as pl
from jax.experimental.pallas import tpu as pltpu
from jax.experimental.pallas import tpu_sc as plsc

_D = 64
_DP = 128
_SUB = 128
_BLK = 256
_SPC = _BLK // _SUB
_NC, _NS = 2, 16
_NW = _NC * _NS
_TBC = 16384
_OBB = 16384


def _tc_pad_transpose(table_t):
    v = table_t.shape[1]

    def body(in_ref, out_ref):
        out_ref[:, :_D] = in_ref[...].T

    grid = (v + _TBC - 1) // _TBC
    return pl.pallas_call(
        body,
        grid=(grid,),
        in_specs=[pl.BlockSpec((_D, _TBC), lambda i: (0, i))],
        out_specs=pl.BlockSpec((_TBC, _DP), lambda i: (i, 0)),
        out_shape=jax.ShapeDtypeStruct((v, _DP), jnp.float32),
    )(table_t)


def _tc_out_transpose(g3, hist, bsz, t0, prev=None):
    th = g3.shape[0]
    out_shape = jax.ShapeDtypeStruct((hist * _D, bsz), jnp.float32)
    g_spec = pl.BlockSpec((1, _OBB, _DP), lambda t, j: (t, j, 0))
    out_spec = pl.BlockSpec((_D, _OBB), lambda t, j: (t0 + t, j))
    grid = (th, bsz // _OBB)

    if prev is None:
        def body0(in_ref, out_ref):
            out_ref[...] = in_ref[0][:, :_D].T

        return pl.pallas_call(
            body0, grid=grid, in_specs=[g_spec], out_specs=out_spec,
            out_shape=out_shape,
        )(g3)

    def body(prev_ref, in_ref, out_ref):
        del prev_ref
        out_ref[...] = in_ref[0][:, :_D].T

    return pl.pallas_call(
        body, grid=grid,
        in_specs=[pl.BlockSpec((8, 128), lambda t, j: (0, 0)), g_spec],
        out_specs=out_spec,
        out_shape=out_shape,
        input_output_aliases={0: 0},
    )(prev, g3)


def _make_emb_body(t0, th):
    def _emb_body(idx_hbm, table_hbm, out_hbm,
                  idx_v, buf0, buf1, gsem0, gsem1, wsem0, wsem1):
        bsz = idx_hbm.shape[1]
        b_per_w = bsz // _NW
        hpb = b_per_w // _BLK
        n_chunks = th * hpb
        wid = lax.axis_index("s") * _NC + lax.axis_index("c")
        b0 = wid * b_per_w

        bufs = (buf0, buf1)
        gsems = (gsem0, gsem1)
        wsems = (wsem0, wsem1)

        pltpu.sync_copy(idx_hbm.at[:, pl.ds(b0, b_per_w)], idx_v)

        def fire(k, slot):
            t = k // hpb
            h = k % hpb
            for j in range(_SPC):
                pltpu.make_async_copy(
                    table_hbm.at[
                        idx_v.at[t0 + t, pl.ds(h * _BLK + j * _SUB, _SUB)]],
                    bufs[slot].at[pl.ds(j * _SUB, _SUB)],
                    gsems[slot],
                ).start()

        def gdrain(slot):
            pltpu.make_async_copy(
                table_hbm.at[pl.ds(0, _BLK)], bufs[slot], gsems[slot]).wait()

        def wdrain(slot):
            pltpu.make_async_copy(
                table_hbm.at[pl.ds(0, _BLK)], bufs[slot], wsems[slot]).wait()

        fire(0, 0)

        def pair_body(k2, carry):
            for s in range(2):
                k = k2 * 2 + s
                t = k // hpb
                bcol = b0 + (k % hpb) * _BLK

                gdrain(s)

                @pl.when(k + 1 < n_chunks)
                def _():
                    @pl.when(k > 0)
                    def _():
                        wdrain(1 - s)
                    fire(k + 1, 1 - s)

                pltpu.make_async_copy(
                    bufs[s],
                    out_hbm.at[pl.ds(t * bsz + bcol, _BLK)],
                    wsems[s],
                ).start()
            return carry

        lax.fori_loop(0, n_chunks // 2, pair_body, 0)
        wdrain(0)
        wdrain(1)

    return _emb_body


def _sc_gather(idx_t, table_pad, t0, th):
    bsz = idx_t.shape[1]
    mesh = plsc.VectorSubcoreMesh(core_axis_name="c", subcore_axis_name="s")
    return pl.kernel(
        _make_emb_body(t0, th),
        out_type=jax.ShapeDtypeStruct((th * bsz, _DP), jnp.float32),
        mesh=mesh,
        scratch_types=[
            pltpu.VMEM((idx_t.shape[0], bsz // _NW), jnp.int32),
            pltpu.VMEM((_BLK, _DP), jnp.float32),
            pltpu.VMEM((_BLK, _DP), jnp.float32),
            pltpu.SemaphoreType.DMA,
            pltpu.SemaphoreType.DMA,
            pltpu.SemaphoreType.DMA,
            pltpu.SemaphoreType.DMA,
        ],
        compiler_params=pltpu.CompilerParams(use_tc_tiling_on_sc=True),
    )(idx_t, table_pad)


def kernel(x, table):
    bsz, hist = x.shape
    n_emb, d = table.shape
    assert d == _D and bsz % (_NW * _BLK) == 0 and hist % 2 == 0
    ha = hist // 2
    idx_t = x.astype(jnp.int32).T
    table_pad = _tc_pad_transpose(table.T)
    g_a = _sc_gather(idx_t, table_pad, 0, ha)
    g_b = _sc_gather(idx_t, table_pad, ha, hist - ha)
    out2 = _tc_out_transpose(g_a.reshape(ha, bsz, _DP), hist, bsz, 0)
    out2 = _tc_out_transpose(
        g_b.reshape(hist - ha, bsz, _DP), hist, bsz, ha, out2)
    return jnp.transpose(out2.reshape(hist, _D, bsz), (2, 0, 1))


# speedup vs baseline: 1.0018x; 1.0018x over previous
import jax
import jax.numpy as jnp
from jax import lax
from jax.experimental import pallas as pl
from jax.experimental.pallas import tpu as pltpu
from jax.experimental.pallas import tpu_sc as plsc

_D = 64
_DP = 128
_SUB = 128
_BLK = 256
_SPC = _BLK // _SUB
_NC, _NS = 2, 16
_NW = _NC * _NS
_TBC = 32768
_OBB = 16384


def _tc_pad_transpose(table_t):
    v = table_t.shape[1]

    def body(in_ref, out_ref):
        out_ref[:, :_D] = in_ref[...].T

    grid = (v + _TBC - 1) // _TBC
    return pl.pallas_call(
        body,
        grid=(grid,),
        in_specs=[pl.BlockSpec((_D, _TBC), lambda i: (0, i))],
        out_specs=pl.BlockSpec((_TBC, _DP), lambda i: (i, 0)),
        out_shape=jax.ShapeDtypeStruct((v, _DP), jnp.float32),
    )(table_t)


def _tc_out_transpose(g3):
    hist, bsz, _ = g3.shape

    def body(in_ref, out_ref):
        out_ref[...] = in_ref[0][:, :_D].T

    return pl.pallas_call(
        body,
        grid=(hist, bsz // _OBB),
        in_specs=[pl.BlockSpec((1, _OBB, _DP), lambda t, j: (t, j, 0))],
        out_specs=pl.BlockSpec((_D, _OBB), lambda t, j: (t, j)),
        out_shape=jax.ShapeDtypeStruct((hist * _D, bsz), jnp.float32),
    )(g3)


def _emb_body(idx_hbm, table_hbm, out_hbm,
              idx_v, buf0, buf1, gsem0, gsem1, wsem0, wsem1):
    hist, bsz = idx_hbm.shape
    b_per_w = bsz // _NW
    hpb = b_per_w // _BLK
    n_chunks = hist * hpb
    wid = lax.axis_index("s") * _NC + lax.axis_index("c")
    b0 = wid * b_per_w

    bufs = (buf0, buf1)
    gsems = (gsem0, gsem1)
    wsems = (wsem0, wsem1)

    pltpu.sync_copy(idx_hbm.at[:, pl.ds(b0, b_per_w)], idx_v)

    def fire(k, slot):
        t = k // hpb
        h = k % hpb
        for j in range(_SPC):
            pltpu.make_async_copy(
                table_hbm.at[idx_v.at[t, pl.ds(h * _BLK + j * _SUB, _SUB)]],
                bufs[slot].at[pl.ds(j * _SUB, _SUB)],
                gsems[slot],
            ).start()

    def gdrain(slot):
        pltpu.make_async_copy(
            table_hbm.at[pl.ds(0, _BLK)], bufs[slot], gsems[slot]).wait()

    def wdrain(slot):
        pltpu.make_async_copy(
            table_hbm.at[pl.ds(0, _BLK)],
            bufs[slot],
            wsems[slot],
        ).wait()

    fire(0, 0)

    def pair_body(k2, carry):
        for s in range(2):
            k = k2 * 2 + s
            t = k // hpb
            bcol = b0 + (k % hpb) * _BLK

            gdrain(s)

            @pl.when(k + 1 < n_chunks)
            def _():
                @pl.when(k > 0)
                def _():
                    wdrain(1 - s)
                fire(k + 1, 1 - s)

            pltpu.make_async_copy(
                bufs[s],
                out_hbm.at[pl.ds(t * bsz + bcol, _BLK)],
                wsems[s],
            ).start()
        return carry

    lax.fori_loop(0, n_chunks // 2, pair_body, 0)
    wdrain(0)
    wdrain(1)


def kernel(x, table):
    bsz, hist = x.shape
    n_emb, d = table.shape
    assert d == _D and bsz % (_NW * _BLK) == 0
    idx_t = x.astype(jnp.int32).T
    table_pad = _tc_pad_transpose(table.T)
    mesh = plsc.VectorSubcoreMesh(core_axis_name="c", subcore_axis_name="s")
    g = pl.kernel(
        _emb_body,
        out_type=jax.ShapeDtypeStruct((hist * bsz, _DP), jnp.float32),
        mesh=mesh,
        scratch_types=[
            pltpu.VMEM((hist, bsz // _NW), jnp.int32),
            pltpu.VMEM((_BLK, _DP), jnp.float32),
            pltpu.VMEM((_BLK, _DP), jnp.float32),
            pltpu.SemaphoreType.DMA,
            pltpu.SemaphoreType.DMA,
            pltpu.SemaphoreType.DMA,
            pltpu.SemaphoreType.DMA,
        ],
        compiler_params=pltpu.CompilerParams(use_tc_tiling_on_sc=True),
    )(idx_t, table_pad)
    out2 = _tc_out_transpose(g.reshape(hist, bsz, _DP))
    return jnp.transpose(out2.reshape(hist, _D, bsz), (2, 0, 1))
